# Initial kernel scaffold; baseline (speedup 1.0000x reference)
import jax
import jax.numpy as jnp
from jax.experimental import pallas as pl


def kernel(SB, PQ, PV, NB, weight, bias):
    raise NotImplementedError("write your pallas kernel here")


# speedup vs baseline: 23.9546x; 23.9546x over previous
import functools

import jax
import jax.numpy as jnp
from jax.experimental import pallas as pl
from jax.experimental.pallas import tpu as pltpu

N = 100000
C = 128
T = 4
EPS = 1e-05
BLK = 2000
NSTEPS = N // BLK


def _stats_body(sb, pq, pv, nb, out, acc):
    i = pl.program_id(0)

    @pl.when(i == 0)
    def _init():
        acc[...] = jnp.zeros_like(acc)

    rows = []
    for ref in (sb, pq, pv, nb):
        x = ref[...]
        xr = x.reshape(BLK // 8, 8, C)
        rows.append(jnp.sum(xr, axis=0))
        rows.append(jnp.sum(xr * xr, axis=0))
    acc[...] += jnp.stack(rows, axis=0)

    @pl.when(i == NSTEPS - 1)
    def _fin():
        out[...] = jnp.sum(acc[...], axis=1)


def _norm_body(stats, w, b, sb, pq, pv, nb, osb, opq, opv, onb):
    sums = stats[0::2, :]
    sqs = stats[1::2, :]
    inv_n = 1.0 / N
    mean = sums * inv_n
    var = sqs * inv_n - mean * mean
    inv_std = jax.lax.rsqrt(jnp.clip(var, EPS, None))
    scale = w[...] * inv_std
    shift = b[...] - mean * scale
    for t, (ref, oref) in enumerate(((sb, osb), (pq, opq), (pv, opv), (nb, onb))):
        oref[...] = ref[...] * scale[t:t + 1, :] + shift[t:t + 1, :]


@jax.jit
def kernel(SB, PQ, PV, NB, weight, bias):
    data_spec = pl.BlockSpec((BLK, C), lambda i: (i, 0))
    stats = pl.pallas_call(
        _stats_body,
        grid=(NSTEPS,),
        in_specs=[data_spec] * 4,
        out_specs=pl.BlockSpec((2 * T, C), lambda i: (0, 0)),
        out_shape=jax.ShapeDtypeStruct((2 * T, C), jnp.float32),
        scratch_shapes=[pltpu.VMEM((2 * T, 8, C), jnp.float32)],
    )(SB, PQ, PV, NB)

    const_spec = pl.BlockSpec((2 * T, C), lambda i: (0, 0))
    wb_spec = pl.BlockSpec((T, C), lambda i: (0, 0))
    outs = pl.pallas_call(
        _norm_body,
        grid=(NSTEPS,),
        in_specs=[const_spec, wb_spec, wb_spec] + [data_spec] * 4,
        out_specs=[data_spec] * 4,
        out_shape=[jax.ShapeDtypeStruct((N, C), jnp.float32)] * 4,
    )(stats, weight, bias, SB, PQ, PV, NB)
    return tuple(outs)
